# baseline (speedup 1.0000x reference)
import functools

import jax
import jax.numpy as jnp
from jax import lax
from jax.experimental import pallas as pl
from jax.experimental.pallas import tpu as pltpu
from jax.experimental.pallas import tpu_sc as plsc

_F = 512
_K = 8
_D = 64
_B = 1024
_C = 4
_T = _F * _C
_W = _F * _K * _C

_NUM_WORKERS = 32
_BPW = _B // _NUM_WORKERS


def _topk_body(crd_ref, eidx_ref):
    x = crd_ref[...]
    xx = jnp.sum(x * x, axis=0, keepdims=True)
    g = lax.dot_general(x, x, (((0,), (0,)), ((), ())))
    d2 = g * (-2.0)
    d2 = d2 + xx
    io_i = lax.broadcasted_iota(jnp.int32, (_F, _F), 0)
    io_j = lax.broadcasted_iota(jnp.int32, (_F, _F), 1)
    xx_col = jnp.sum(
        jnp.where(io_i == io_j, jnp.broadcast_to(xx, (_F, _F)), 0.0),
        axis=1, keepdims=True)
    d2 = d2 + xx_col
    dist = jnp.sqrt(jnp.maximum(d2, 0.0))
    col4 = lax.broadcasted_iota(jnp.int32, (_F, _C), 1)
    row4 = lax.broadcasted_iota(jnp.int32, (_F, _C), 0)
    big_i = jnp.int32(1 << 30)
    inf = jnp.float32(jnp.inf)
    for n in range(_K):
        m = jnp.min(dist, axis=1, keepdims=True)
        cand = jnp.where(dist == m, io_j, big_i)
        sel = jnp.min(cand, axis=1, keepdims=True)
        dist = jnp.where(io_j == sel, inf, dist)
        v = sel * 4 + col4
        if n == 0:
            v = jnp.where(row4 == 0, col4, v)
        eidx_ref[:, n * _C:(n + 1) * _C] = v


_topk_call = pl.pallas_call(
    _topk_body,
    out_shape=jax.ShapeDtypeStruct((_F, _K * _C), jnp.int32),
)


def _gather_body(x_hbm, eidx_hbm, out_hbm, eidx_v, table_v, row_v):
    wid = lax.axis_index("s") * 2 + lax.axis_index("c")
    b0 = wid * _BPW
    pltpu.sync_copy(eidx_hbm, eidx_v)

    def body(i, carry):
        b = b0 + i
        pltpu.sync_copy(x_hbm.at[b], table_v)

        def inner(j, c2):
            v = eidx_v[pl.ds(j * 16, 16)]
            row_v[pl.ds(j * 16, 16)] = plsc.load_gather(table_v, [v])
            return c2

        lax.fori_loop(0, _W // 16, inner, 0)
        pltpu.sync_copy(row_v, out_hbm.at[b])
        return carry

    lax.fori_loop(0, _BPW, body, 0)


_gather_call = functools.partial(
    pl.kernel,
    mesh=plsc.VectorSubcoreMesh(core_axis_name="c", subcore_axis_name="s"),
    out_type=jax.ShapeDtypeStruct((_B, _W), jnp.float32),
    scratch_types=[
        pltpu.VMEM((_W,), jnp.int32),
        pltpu.VMEM((_T,), jnp.float32),
        pltpu.VMEM((_W,), jnp.float32),
    ],
    compiler_params=pltpu.CompilerParams(needs_layout_passes=False),
)(_gather_body)


def kernel(coordinates, inputs):
    crd = coordinates.reshape(coordinates.shape[0], coordinates.shape[2])
    eidx = _topk_call(crd)
    x2 = inputs.reshape(_B, _T)
    out2 = _gather_call(x2, eidx.reshape(_W))
    return out2.reshape(_B, 1, _F * _K, _C)


# speedup vs baseline: 1.7225x; 1.7225x over previous
import functools

import jax
import jax.numpy as jnp
from jax import lax
from jax.experimental import pallas as pl
from jax.experimental.pallas import tpu as pltpu
from jax.experimental.pallas import tpu_sc as plsc

_F = 512
_K = 8
_D = 64
_B = 1024
_C = 4
_T = _F * _C
_W = _F * _K * _C

_NUM_WORKERS = 32
_BPW = _B // _NUM_WORKERS


def _topk_body(crd_ref, eidx_ref):
    x = crd_ref[...]
    xx = jnp.sum(x * x, axis=0, keepdims=True)
    g = lax.dot_general(x, x, (((0,), (0,)), ((), ())))
    d2 = g * (-2.0)
    d2 = d2 + xx
    io_i = lax.broadcasted_iota(jnp.int32, (_F, _F), 0)
    io_j = lax.broadcasted_iota(jnp.int32, (_F, _F), 1)
    xx_col = jnp.sum(
        jnp.where(io_i == io_j, jnp.broadcast_to(xx, (_F, _F)), 0.0),
        axis=1, keepdims=True)
    d2 = d2 + xx_col
    dist = jnp.sqrt(jnp.maximum(d2, 0.0))
    col4 = lax.broadcasted_iota(jnp.int32, (_F, _C), 1)
    row4 = lax.broadcasted_iota(jnp.int32, (_F, _C), 0)
    big_i = jnp.int32(1 << 30)
    inf = jnp.float32(jnp.inf)
    for n in range(_K):
        m = jnp.min(dist, axis=1, keepdims=True)
        cand = jnp.where(dist == m, io_j, big_i)
        sel = jnp.min(cand, axis=1, keepdims=True)
        dist = jnp.where(io_j == sel, inf, dist)
        v = sel * 4 + col4
        if n == 0:
            v = jnp.where(row4 == 0, col4, v)
        eidx_ref[:, n * _C:(n + 1) * _C] = v


_topk_call = pl.pallas_call(
    _topk_body,
    out_shape=jax.ShapeDtypeStruct((_F, _K * _C), jnp.int32),
)


def _gather_body(x_hbm, eidx_hbm, out_hbm, eidx_v, table_v, row0, row1,
                 sem_t, sem_e, sem_o0, sem_o1):
    wid = lax.axis_index("s") * 2 + lax.axis_index("c")
    b0 = wid * _BPW
    cp_t = pltpu.async_copy(x_hbm.at[pl.ds(b0, _BPW)], table_v, sem_t)
    cp_e = pltpu.async_copy(eidx_hbm, eidx_v, sem_e)
    cp_t.wait()
    cp_e.wait()

    def run_batch(row, bvec):
        @plsc.parallel_loop(0, _W // 16, unroll=8)
        def chunk(j):
            v = eidx_v[pl.ds(j * 16, 16)]
            row[pl.ds(j * 16, 16)] = plsc.load_gather(table_v, [bvec, v])

    rows = (row0, row1)
    sems = (sem_o0, sem_o1)
    pending = [None, None]
    for b in range(_BPW):
        s = b & 1
        if pending[s] is not None:
            pending[s].wait()
        run_batch(rows[s], jnp.full((16,), b, jnp.int32))
        pending[s] = pltpu.async_copy(rows[s], out_hbm.at[b0 + b], sems[s])
    pending[0].wait()
    pending[1].wait()


_gather_call = functools.partial(
    pl.kernel,
    mesh=plsc.VectorSubcoreMesh(core_axis_name="c", subcore_axis_name="s"),
    out_type=jax.ShapeDtypeStruct((_B, _W), jnp.float32),
    scratch_types=[
        pltpu.VMEM((_W,), jnp.int32),
        pltpu.VMEM((_BPW, _T), jnp.float32),
        pltpu.VMEM((_W,), jnp.float32),
        pltpu.VMEM((_W,), jnp.float32),
        pltpu.SemaphoreType.DMA,
        pltpu.SemaphoreType.DMA,
        pltpu.SemaphoreType.DMA,
        pltpu.SemaphoreType.DMA,
    ],
    compiler_params=pltpu.CompilerParams(needs_layout_passes=False),
)(_gather_body)


def kernel(coordinates, inputs):
    crd = coordinates.reshape(coordinates.shape[0], coordinates.shape[2])
    eidx = _topk_call(crd)
    x2 = inputs.reshape(_B, _T)
    out2 = _gather_call(x2, eidx.reshape(_W))
    return out2.reshape(_B, 1, _F * _K, _C)


# speedup vs baseline: 6.2491x; 3.6279x over previous
import functools

import jax
import jax.numpy as jnp
from jax import lax
from jax.experimental import pallas as pl
from jax.experimental.pallas import tpu as pltpu
from jax.experimental.pallas import tpu_sc as plsc

_F = 512
_K = 8
_B = 1024
_C = 4
_T = _F * _C
_W = _F * _K * _C

_NUM_WORKERS = 32
_BPW = _B // _NUM_WORKERS


def _topk_body(crd_ref, idx_ref):
    x = crd_ref[...]
    xx = jnp.sum(x * x, axis=0, keepdims=True)
    g = lax.dot_general(x, x, (((0,), (0,)), ((), ())))
    d2 = g * (-2.0)
    d2 = d2 + xx
    io_i = lax.broadcasted_iota(jnp.int32, (_F, _F), 0)
    io_j = lax.broadcasted_iota(jnp.int32, (_F, _F), 1)
    xx_col = jnp.sum(
        jnp.where(io_i == io_j, jnp.broadcast_to(xx, (_F, _F)), 0.0),
        axis=1, keepdims=True)
    d2 = d2 + xx_col
    dist = jnp.sqrt(jnp.maximum(d2, 0.0))
    col1 = lax.broadcasted_iota(jnp.int32, (1, _F), 1)
    big_i = jnp.int32(1 << 30)
    inf = jnp.float32(jnp.inf)
    for n in range(_K):
        m = jnp.min(dist, axis=0, keepdims=True)
        cand = jnp.where(dist == m, io_i, big_i)
        sel = jnp.min(cand, axis=0, keepdims=True)
        dist = jnp.where(io_i == sel, inf, dist)
        if n == 0:
            sel = jnp.where(col1 == 0, 0, sel)
        idx_ref[pl.ds(n * _F, _F)] = sel.reshape(_F)


_topk_call = pl.pallas_call(
    _topk_body,
    out_shape=jax.ShapeDtypeStruct((_K * _F,), jnp.int32),
)


def _gather_body(x_hbm, idx_hbm, out_hbm, idx_v, eidx_v, table_v, row0, row1,
                 sem_t, sem_i, sem_o0, sem_o1):
    wid = lax.axis_index("s") * 2 + lax.axis_index("c")
    b0 = wid * _BPW
    cp_t = pltpu.async_copy(x_hbm.at[pl.ds(b0 * _T, _BPW * _T)], table_v, sem_t)
    cp_i = pltpu.async_copy(idx_hbm, idx_v, sem_i)
    cp_i.wait()

    lanes = lax.iota(jnp.int32, 16)

    @plsc.parallel_loop(0, _W // 16, unroll=4)
    def build(t):
        jbase = ((t >> 5) << 7) + ((t & 7) << 4)
        c = (t >> 3) & 3
        jv = jbase + lanes
        pos = ((jv & 7) << 9) + (jv >> 3)
        f = plsc.load_gather(idx_v, [pos])
        e = ((f >> 7) << 9) + (c << 7) + (f & 127)
        eidx_v[pl.ds(t * 16, 16)] = e

    cp_t.wait()
    rows = (row0, row1)
    sems = (sem_o0, sem_o1)
    pending = [None, None]
    for b in range(_BPW):
        s = b & 1
        if pending[s] is not None:
            pending[s].wait()
        row = rows[s]
        off = jnp.int32(b * _T)

        @plsc.parallel_loop(0, _W // 16, unroll=8)
        def chunk(t, row=row, off=off):
            v = eidx_v[pl.ds(t * 16, 16)] + off
            row[pl.ds(t * 16, 16)] = plsc.load_gather(table_v, [v])

        pending[s] = pltpu.async_copy(
            row, out_hbm.at[pl.ds((b0 + b) * _W, _W)], sems[s])
    pending[0].wait()
    pending[1].wait()


_gather_call = functools.partial(
    pl.kernel,
    mesh=plsc.VectorSubcoreMesh(core_axis_name="c", subcore_axis_name="s"),
    out_type=jax.ShapeDtypeStruct((_B * _W,), jnp.float32),
    scratch_types=[
        pltpu.VMEM((_K * _F,), jnp.int32),
        pltpu.VMEM((_W,), jnp.int32),
        pltpu.VMEM((_BPW * _T,), jnp.float32),
        pltpu.VMEM((_W,), jnp.float32),
        pltpu.VMEM((_W,), jnp.float32),
        pltpu.SemaphoreType.DMA,
        pltpu.SemaphoreType.DMA,
        pltpu.SemaphoreType.DMA,
        pltpu.SemaphoreType.DMA,
    ],
    compiler_params=pltpu.CompilerParams(needs_layout_passes=False),
)(_gather_body)


def kernel(coordinates, inputs):
    crd = coordinates.reshape(coordinates.shape[0], coordinates.shape[2])
    idx1d = _topk_call(crd)
    xb = inputs.reshape(_B, 4, 128, _C).transpose(0, 1, 3, 2).reshape(_B * _T)
    out1d = _gather_call(xb, idx1d)
    out = (out1d.reshape(_B, _W // 512, _C, 128)
           .transpose(0, 1, 3, 2)
           .reshape(_B, 1, _F * _K, _C))
    return out


# speedup vs baseline: 7.2006x; 1.1523x over previous
import functools

import jax
import jax.numpy as jnp
from jax import lax
from jax.experimental import pallas as pl
from jax.experimental.pallas import tpu as pltpu
from jax.experimental.pallas import tpu_sc as plsc

_F = 512
_K = 8
_B = 1024
_C = 4
_T = _F * _C
_W = _F * _K * _C

_NUM_WORKERS = 32
_BPW = _B // _NUM_WORKERS


def _topk_body(crd_ref, idx_ref):
    x = crd_ref[...]
    xx = jnp.sum(x * x, axis=0, keepdims=True)
    g = lax.dot_general(x, x, (((0,), (0,)), ((), ())))
    d2 = g * (-2.0)
    d2 = d2 + xx
    io_i = lax.broadcasted_iota(jnp.int32, (_F, _F), 0)
    io_j = lax.broadcasted_iota(jnp.int32, (_F, _F), 1)
    xx_col = jnp.sum(
        jnp.where(io_i == io_j, jnp.broadcast_to(xx, (_F, _F)), 0.0),
        axis=1, keepdims=True)
    d2 = d2 + xx_col
    dist = jnp.sqrt(jnp.maximum(d2, 0.0))
    col1 = lax.broadcasted_iota(jnp.int32, (1, _F), 1)
    big_i = jnp.int32(1 << 30)
    inf = jnp.float32(jnp.inf)
    for n in range(_K):
        m = jnp.min(dist, axis=0, keepdims=True)
        cand = jnp.where(dist == m, io_i, big_i)
        sel = jnp.min(cand, axis=0, keepdims=True)
        dist = jnp.where(io_i == sel, inf, dist)
        if n == 0:
            sel = jnp.where(col1 == 0, 0, sel)
        idx_ref[pl.ds(n * _F, _F)] = sel.reshape(_F)


_topk_call = pl.pallas_call(
    _topk_body,
    out_shape=jax.ShapeDtypeStruct((_K * _F,), jnp.int32),
)


_G = 8


def _gather_body(x_hbm, idx_hbm, out_hbm, idx_v, eidx_v, tbl0, tbl1,
                 r0, r1, r2, r3, sem_t0, sem_t1, sem_i,
                 so0, so1, so2, so3):
    wid = lax.axis_index("s") * 2 + lax.axis_index("c")
    b0 = wid * _BPW
    tbls = (tbl0, tbl1)
    tsems = (sem_t0, sem_t1)
    cp_t = [pltpu.async_copy(x_hbm.at[pl.ds(b0 * _T, _G * _T)], tbl0, sem_t0),
            None]
    cp_i = pltpu.async_copy(idx_hbm, idx_v, sem_i)
    cp_i.wait()

    lanes = lax.iota(jnp.int32, 16)

    @plsc.parallel_loop(0, _W // 16, unroll=4)
    def build(t):
        jbase = ((t >> 5) << 7) + ((t & 7) << 4)
        c = (t >> 3) & 3
        jv = jbase + lanes
        pos = ((jv & 7) << 9) + (jv >> 3)
        f = plsc.load_gather(idx_v, [pos])
        e = ((f >> 7) << 9) + (c << 7) + (f & 127)
        eidx_v[pl.ds(t * 16, 16)] = e

    rows = (r0, r1, r2, r3)
    rsems = (so0, so1, so2, so3)
    pending = [None] * 4
    ngroups = _BPW // _G
    for g in range(ngroups):
        tg = g & 1
        if g + 1 < ngroups:
            cp_t[1 - tg] = pltpu.async_copy(
                x_hbm.at[pl.ds((b0 + (g + 1) * _G) * _T, _G * _T)],
                tbls[1 - tg], tsems[1 - tg])
        cp_t[tg].wait()
        tbl = tbls[tg]
        for p in range(_G // 2):
            q = g * (_G // 2) + p
            sa, sb = 2 * (q & 1), 2 * (q & 1) + 1
            if pending[sa] is not None:
                pending[sa].wait()
            if pending[sb] is not None:
                pending[sb].wait()
            ra, rb = rows[sa], rows[sb]
            offa = jnp.int32(2 * p * _T)
            offb = jnp.int32((2 * p + 1) * _T)

            @plsc.parallel_loop(0, _W // 16, unroll=8)
            def chunk(t, ra=ra, rb=rb, offa=offa, offb=offb, tbl=tbl):
                v = eidx_v[pl.ds(t * 16, 16)]
                ra[pl.ds(t * 16, 16)] = plsc.load_gather(tbl, [v + offa])
                rb[pl.ds(t * 16, 16)] = plsc.load_gather(tbl, [v + offb])

            ba = b0 + g * _G + 2 * p
            pending[sa] = pltpu.async_copy(
                ra, out_hbm.at[pl.ds(ba * _W, _W)], rsems[sa])
            pending[sb] = pltpu.async_copy(
                rb, out_hbm.at[pl.ds((ba + 1) * _W, _W)], rsems[sb])
    for s in range(4):
        if pending[s] is not None:
            pending[s].wait()


_gather_call = functools.partial(
    pl.kernel,
    mesh=plsc.VectorSubcoreMesh(core_axis_name="c", subcore_axis_name="s"),
    out_type=jax.ShapeDtypeStruct((_B * _W,), jnp.float32),
    scratch_types=[
        pltpu.VMEM((_K * _F,), jnp.int32),
        pltpu.VMEM((_W,), jnp.int32),
        pltpu.VMEM((_G * _T,), jnp.float32),
        pltpu.VMEM((_G * _T,), jnp.float32),
        pltpu.VMEM((_W,), jnp.float32),
        pltpu.VMEM((_W,), jnp.float32),
        pltpu.VMEM((_W,), jnp.float32),
        pltpu.VMEM((_W,), jnp.float32),
        pltpu.SemaphoreType.DMA,
        pltpu.SemaphoreType.DMA,
        pltpu.SemaphoreType.DMA,
        pltpu.SemaphoreType.DMA,
        pltpu.SemaphoreType.DMA,
        pltpu.SemaphoreType.DMA,
        pltpu.SemaphoreType.DMA,
    ],
    compiler_params=pltpu.CompilerParams(needs_layout_passes=False),
)(_gather_body)


def kernel(coordinates, inputs):
    crd = coordinates.reshape(coordinates.shape[0], coordinates.shape[2])
    idx1d = _topk_call(crd)
    xb = inputs.reshape(_B, 4, 128, _C).transpose(0, 1, 3, 2).reshape(_B * _T)
    out1d = _gather_call(xb, idx1d)
    out = (out1d.reshape(_B, _W // 512, _C, 128)
           .transpose(0, 1, 3, 2)
           .reshape(_B, 1, _F * _K, _C))
    return out


# speedup vs baseline: 7.8542x; 1.0908x over previous
import functools

import jax
import jax.numpy as jnp
from jax import lax
from jax.experimental import pallas as pl
from jax.experimental.pallas import tpu as pltpu
from jax.experimental.pallas import tpu_sc as plsc

_F = 512
_K = 8
_B = 1024
_C = 4
_T = _F * _C
_W = _F * _K * _C

_NUM_WORKERS = 32
_BPW = _B // _NUM_WORKERS


def _topk_body(crd_ref, idx_ref):
    x = crd_ref[...]
    xx = jnp.sum(x * x, axis=0, keepdims=True)
    g = lax.dot_general(x, x, (((0,), (0,)), ((), ())))
    d2 = g * (-2.0)
    d2 = d2 + xx
    io_i = lax.broadcasted_iota(jnp.int32, (_F, _F), 0)
    io_j = lax.broadcasted_iota(jnp.int32, (_F, _F), 1)
    xx_col = jnp.sum(
        jnp.where(io_i == io_j, jnp.broadcast_to(xx, (_F, _F)), 0.0),
        axis=1, keepdims=True)
    d2 = d2 + xx_col
    dist = jnp.sqrt(jnp.maximum(d2, 0.0))
    col1 = lax.broadcasted_iota(jnp.int32, (1, _F), 1)
    big_i = jnp.int32(1 << 30)
    inf = jnp.float32(jnp.inf)
    for n in range(_K):
        m = jnp.min(dist, axis=0, keepdims=True)
        cand = jnp.where(dist == m, io_i, big_i)
        sel = jnp.min(cand, axis=0, keepdims=True)
        dist = jnp.where(io_i == sel, inf, dist)
        if n == 0:
            sel = jnp.where(col1 == 0, 0, sel)
        idx_ref[pl.ds(n * _F, _F)] = sel.reshape(_F)


_topk_call = pl.pallas_call(
    _topk_body,
    out_shape=jax.ShapeDtypeStruct((_K * _F,), jnp.int32),
)


_G = 8


def _gather_body(x_hbm, idx_hbm, out_hbm, idx_v, eidx_v, tbl0, tbl1,
                 r0, r1, r2, r3, sem_t0, sem_t1, sem_i,
                 so0, so1, so2, so3):
    wid = lax.axis_index("s") * 2 + lax.axis_index("c")
    b0 = wid * _BPW
    tbls = (tbl0, tbl1)
    tsems = (sem_t0, sem_t1)
    cp_t = [pltpu.async_copy(x_hbm.at[pl.ds(b0 * _T, _G * _T)], tbl0, sem_t0),
            None]
    cp_i = pltpu.async_copy(idx_hbm, idx_v, sem_i)
    cp_i.wait()

    lanes = lax.iota(jnp.int32, 16)

    @plsc.parallel_loop(0, _W // 64, unroll=8)
    def build(s):
        jv = s * 16 + lanes
        pos = ((jv & 7) << 9) + (jv >> 3)
        f = plsc.load_gather(idx_v, [pos])
        e = ((f >> 7) << 9) + (f & 127)
        eidx_v[pl.ds(s * 16, 16)] = e

    rows = (r0, r1, r2, r3)
    rsems = (so0, so1, so2, so3)
    pending = [None] * 4
    ngroups = _BPW // _G
    for g in range(ngroups):
        tg = g & 1
        if g + 1 < ngroups:
            cp_t[1 - tg] = pltpu.async_copy(
                x_hbm.at[pl.ds((b0 + (g + 1) * _G) * _T, _G * _T)],
                tbls[1 - tg], tsems[1 - tg])
        cp_t[tg].wait()
        tbl = tbls[tg]
        for p in range(_G // 2):
            q = g * (_G // 2) + p
            sa, sb = 2 * (q & 1), 2 * (q & 1) + 1
            if pending[sa] is not None:
                pending[sa].wait()
            if pending[sb] is not None:
                pending[sb].wait()
            ra, rb = rows[sa], rows[sb]
            offa = 2 * p * _T
            offb = (2 * p + 1) * _T

            @plsc.parallel_loop(0, _W // 64, unroll=2)
            def chunk(s, ra=ra, rb=rb, offa=offa, offb=offb, tbl=tbl):
                v = eidx_v[pl.ds(s * 16, 16)]
                wbase = s * 16 + (s >> 3) * 384
                for c in range(_C):
                    w = wbase + c * 128
                    ra[pl.ds(w, 16)] = plsc.load_gather(
                        tbl, [v + jnp.int32(offa + c * 128)])
                    rb[pl.ds(w, 16)] = plsc.load_gather(
                        tbl, [v + jnp.int32(offb + c * 128)])

            ba = b0 + g * _G + 2 * p
            pending[sa] = pltpu.async_copy(
                ra, out_hbm.at[pl.ds(ba * _W, _W)], rsems[sa])
            pending[sb] = pltpu.async_copy(
                rb, out_hbm.at[pl.ds((ba + 1) * _W, _W)], rsems[sb])
    for s in range(4):
        if pending[s] is not None:
            pending[s].wait()


_gather_call = functools.partial(
    pl.kernel,
    mesh=plsc.VectorSubcoreMesh(core_axis_name="c", subcore_axis_name="s"),
    out_type=jax.ShapeDtypeStruct((_B * _W,), jnp.float32),
    scratch_types=[
        pltpu.VMEM((_K * _F,), jnp.int32),
        pltpu.VMEM((_W // 4,), jnp.int32),
        pltpu.VMEM((_G * _T,), jnp.float32),
        pltpu.VMEM((_G * _T,), jnp.float32),
        pltpu.VMEM((_W,), jnp.float32),
        pltpu.VMEM((_W,), jnp.float32),
        pltpu.VMEM((_W,), jnp.float32),
        pltpu.VMEM((_W,), jnp.float32),
        pltpu.SemaphoreType.DMA,
        pltpu.SemaphoreType.DMA,
        pltpu.SemaphoreType.DMA,
        pltpu.SemaphoreType.DMA,
        pltpu.SemaphoreType.DMA,
        pltpu.SemaphoreType.DMA,
        pltpu.SemaphoreType.DMA,
    ],
    compiler_params=pltpu.CompilerParams(needs_layout_passes=False),
)(_gather_body)


def kernel(coordinates, inputs):
    crd = coordinates.reshape(coordinates.shape[0], coordinates.shape[2])
    idx1d = _topk_call(crd)
    xb = inputs.reshape(_B, 4, 128, _C).transpose(0, 1, 3, 2).reshape(_B * _T)
    out1d = _gather_call(xb, idx1d)
    out = (out1d.reshape(_B, _W // 512, _C, 128)
           .transpose(0, 1, 3, 2)
           .reshape(_B, 1, _F * _K, _C))
    return out
